# baseline (speedup 1.0000x reference)
import functools

import jax
import jax.numpy as jnp
from jax import lax
from jax.experimental import pallas as pl
from jax.experimental.pallas import tpu as pltpu
from jax.experimental.pallas import tpu_sc as plsc

_FM = 128
_NV = 10000
_NCON = 10000
_NE = 320000
_OUT = 16

_NSC = 2
_NSUB = 16
_EPS = _NE // _NSUB
_CH = 128
_NCHUNK = 159
_EPSP = _NCHUNK * _CH
_RPC = 5120
_NPAD = _NSC * _RPC
_ACC = _RPC + 8
_RPS = _RPC // _NSUB

_DNUMS = lax.GatherDimensionNumbers(
    offset_dims=(), collapsed_slice_dims=(0,), start_index_map=(0,))


def _segsum_body(table, packed, zeros, out,
                 pk0, pk1, pk2, rows0, rows1, rows2, sloc0, sloc1, sloc2,
                 acc, gsem0, gsem1, gsem2, psem0, psem1, psem2,
                 ssem0, ssem1, ssem2):
    c = lax.axis_index("c")
    s = lax.axis_index("s")
    base = c * _RPC
    pk = (pk0, pk1, pk2)
    rows = (rows0, rows1, rows2)
    sloc = (sloc0, sloc1, sloc2)
    gsem = (gsem0, gsem1, gsem2)
    psem = (psem0, psem1, psem2)
    ssem = (ssem0, ssem1, ssem2)

    pltpu.sync_copy(zeros, acc.at[pl.ds(s * _RPS, _RPS)])
    pltpu.sync_copy(packed.at[s, 0], pk0)
    pltpu.async_copy(packed.at[s, 1], pk1, psem1)
    pltpu.async_copy(table.at[pk0.at[0]], rows0, gsem0)
    pltpu.async_copy(zeros.at[pl.ds(0, _CH)], rows1, ssem1)
    pltpu.async_copy(zeros.at[pl.ds(0, _CH)], rows2, ssem2)
    plsc.subcore_barrier()

    def step(t, p):
        q = (p + 1) % 3
        o = (p + 2) % 3
        tn = jnp.minimum(t + 1, _NCHUNK - 1)
        tnn = jnp.minimum(t + 2, _NCHUNK - 1)
        pltpu.make_async_copy(packed.at[s, tn], pk[q], psem[q]).wait()
        pltpu.make_async_copy(zeros.at[pl.ds(0, _CH)], rows[q],
                              ssem[q]).wait()
        pltpu.async_copy(table.at[pk[q].at[0]], rows[q], gsem[q])
        pltpu.make_async_copy(table.at[pl.ds(0, _CH)], rows[p],
                              gsem[p]).wait()
        pltpu.async_copy(packed.at[s, tnn], pk[o], psem[o])

        def grp(g, carry2):
            i16 = pk[p][1, pl.ds(g * 16, 16)] - base
            ok = jnp.logical_and(i16 >= 0, i16 < _RPC)
            sloc[p][pl.ds(g * 16, 16)] = jnp.where(ok, i16, _RPC)
            v16 = lax.bitcast_convert_type(
                pk[p][2, pl.ds(g * 16, 16)], jnp.float32)
            for j in range(16):
                b16 = lax.gather(
                    v16, jnp.full((16, 1), j, jnp.int32), _DNUMS, (1,),
                    mode=lax.GatherScatterMode.PROMISE_IN_BOUNDS)
                r = g * 16 + j
                for k in range(8):
                    rows[p][r, pl.ds(k * 16, 16)] = (
                        rows[p][r, pl.ds(k * 16, 16)] * b16)
            return carry2

        lax.fori_loop(0, _CH // 16, grp, 0)
        pltpu.async_copy(rows[p], acc.at[sloc[p]], ssem[p], add=True)

    def tri(t3, carry):
        step(t3 * 3, 0)
        step(t3 * 3 + 1, 1)
        step(t3 * 3 + 2, 2)
        return carry

    lax.fori_loop(0, _NCHUNK // 3, tri, 0)
    pltpu.make_async_copy(table.at[pl.ds(0, _CH)], rows0, gsem0).wait()
    pltpu.make_async_copy(packed.at[s, 0], pk1, psem1).wait()
    pltpu.make_async_copy(zeros.at[pl.ds(0, _CH)], rows1, ssem1).wait()
    pltpu.make_async_copy(zeros.at[pl.ds(0, _CH)], rows2, ssem2).wait()
    plsc.subcore_barrier()
    pltpu.sync_copy(acc.at[pl.ds(s * _RPS, _RPS)],
                    out.at[pl.ds(base + s * _RPS, _RPS)])


_segsum_sc = functools.partial(
    pl.kernel,
    mesh=plsc.VectorSubcoreMesh(core_axis_name="c", subcore_axis_name="s"),
    out_type=jax.ShapeDtypeStruct((_NPAD, _FM), jnp.float32),
    scratch_types=[
        pltpu.VMEM((3, _CH), jnp.int32),
        pltpu.VMEM((3, _CH), jnp.int32),
        pltpu.VMEM((3, _CH), jnp.int32),
        pltpu.VMEM((_CH, _FM), jnp.float32),
        pltpu.VMEM((_CH, _FM), jnp.float32),
        pltpu.VMEM((_CH, _FM), jnp.float32),
        pltpu.VMEM((_CH,), jnp.int32),
        pltpu.VMEM((_CH,), jnp.int32),
        pltpu.VMEM((_CH,), jnp.int32),
        pltpu.VMEM_SHARED((_ACC, _FM), jnp.float32),
        pltpu.SemaphoreType.DMA,
        pltpu.SemaphoreType.DMA,
        pltpu.SemaphoreType.DMA,
        pltpu.SemaphoreType.DMA,
        pltpu.SemaphoreType.DMA,
        pltpu.SemaphoreType.DMA,
        pltpu.SemaphoreType.DMA,
        pltpu.SemaphoreType.DMA,
        pltpu.SemaphoreType.DMA,
    ],
)(_segsum_body)


def _mm(x, W, b):
    return jnp.dot(x, W, preferred_element_type=jnp.float32) + b


def _pairnorm(y):
    y = y - jnp.mean(y, axis=0, keepdims=True)
    rn = jnp.sqrt(1e-6 + jnp.mean(jnp.sum(y * y, axis=1)))
    return y / rn


def _q_body(var_ref, qc1W, qc1b, qc2W, qc2b, qo1W, qo1b, qo2W, qo2b, objm,
            cq_ref, oq_ref):
    v = var_ref[...]
    h = jnp.maximum(_mm(v, qc1W[...], qc1b[...]), 0.0)
    cq_ref[...] = _mm(h, qc2W[...], qc2b[...])
    h = jnp.maximum(_mm(v, qo1W[...], qo1b[...]), 0.0)
    oq_ref[...] = _mm(h, qo2W[...], qo2b[...]) * objm[...]


def _c_body(con_ref, v2c_ref, cv_ref, cu1aW, cu1bW, cu1b, cu2W, cu2b,
            out_ref):
    v2c = v2c_ref[:_NCON]
    loss = jnp.maximum(v2c - cv_ref[...], 0.0)
    h = jnp.dot(con_ref[...], cu1aW[...], preferred_element_type=jnp.float32)
    h = h + jnp.dot(loss, cu1bW[...], preferred_element_type=jnp.float32)
    h = jnp.maximum(h + cu1b[...], 0.0)
    out_ref[...] = _pairnorm(_mm(h, cu2W[...], cu2b[...]))


def _v_body(var_ref, c2v_ref, oq_ref, vu1aW, vu1bW, vu1cW, vu1b, vu2W, vu2b,
            o1W, o1b, o2W, o2b, nim_ref, newvar_ref, out_ref):
    h = jnp.dot(var_ref[...], vu1aW[...], preferred_element_type=jnp.float32)
    h = h + jnp.dot(c2v_ref[:_NV], vu1bW[...],
                    preferred_element_type=jnp.float32)
    h = h + jnp.dot(oq_ref[...], vu1cW[...], preferred_element_type=jnp.float32)
    h = jnp.maximum(h + vu1b[...], 0.0)
    y = _pairnorm(_mm(h, vu2W[...], vu2b[...]))
    newvar_ref[...] = y
    h = jnp.maximum(_mm(y, o1W[...], o1b[...]), 0.0)
    out_ref[...] = jax.nn.sigmoid(_mm(h, o2W[...], o2b[...]) + nim_ref[...])


def _f32(shape):
    return jax.ShapeDtypeStruct(shape, jnp.float32)


def _pack_edges(gi, si, va):
    pad = _EPSP - _EPS
    g = jnp.pad(gi.reshape(_NSUB, _EPS), ((0, 0), (0, pad)))
    sd = jnp.pad(si.reshape(_NSUB, _EPS), ((0, 0), (0, pad)))
    vb = lax.bitcast_convert_type(va, jnp.int32)
    v = jnp.pad(vb.reshape(_NSUB, _EPS), ((0, 0), (0, pad)))
    return jnp.stack([g.reshape(_NSUB, _NCHUNK, _CH),
                      sd.reshape(_NSUB, _NCHUNK, _CH),
                      v.reshape(_NSUB, _NCHUNK, _CH)], axis=2)


def kernel(edge_index, edge_vals, objective_multipliers, const_values,
           integer_mask, cu1_W, cu1_b, cu2_W, cu2_b, qc1_W, qc1_b, qc2_W,
           qc2_b, qo1_W, qo1_b, qo2_W, qo2_b, vu1_W, vu1_b, vu2_W, vu2_b,
           o1_W, o1_b, o2_W, o2_b):
    packed_v2c = _pack_edges(edge_index[0], edge_index[1], edge_vals)
    packed_c2v = _pack_edges(edge_index[1], edge_index[0], edge_vals)
    zeros = jnp.zeros((_RPS, _FM), dtype=jnp.float32)
    objm = objective_multipliers[:, None]
    cv = const_values[:, None]
    im = integer_mask[:, None]
    b = {n: v.reshape(1, -1) for n, v in (
        ("cu1", cu1_b), ("cu2", cu2_b), ("qc1", qc1_b), ("qc2", qc2_b),
        ("qo1", qo1_b), ("qo2", qo2_b), ("vu1", vu1_b), ("vu2", vu2_b),
        ("o1", o1_b), ("o2", o2_b))}

    q_call = pl.pallas_call(
        _q_body, out_shape=[_f32((_NV, _FM)), _f32((_NV, _FM))])
    c_call = pl.pallas_call(_c_body, out_shape=_f32((_NCON, _FM)))
    v_call = pl.pallas_call(
        _v_body, out_shape=[_f32((_NV, _FM)), _f32((_NV, _OUT))])

    variables = jnp.ones((_NV, _FM), dtype=jnp.float32)
    constraints = jnp.ones((_NCON, _FM), dtype=jnp.float32)
    nkey = jax.random.key(42)
    outputs = []
    for i in range(3):
        cq, oq = q_call(variables, qc1_W, b["qc1"], qc2_W, b["qc2"],
                        qo1_W, b["qo1"], qo2_W, b["qo2"], objm)
        v2c = _segsum_sc(cq, packed_v2c, zeros)
        constraints = c_call(constraints, v2c, cv, cu1_W[:_FM],
                             cu1_W[_FM:], b["cu1"], cu2_W, b["cu2"])
        c2v = _segsum_sc(constraints, packed_c2v, zeros)
        noise = jax.random.normal(jax.random.fold_in(nkey, i), (_NV, _OUT),
                                  dtype=jnp.float32)
        variables, out_i = v_call(
            variables, c2v, oq, vu1_W[:_FM], vu1_W[_FM:2 * _FM],
            vu1_W[2 * _FM:], b["vu1"], vu2_W, b["vu2"], o1_W, b["o1"],
            o2_W, b["o2"], noise * im)
        outputs.append(out_i)
    return jnp.stack(outputs)


# speedup vs baseline: 1.7551x; 1.7551x over previous
import functools

import jax
import jax.numpy as jnp
from jax import lax
from jax.experimental import pallas as pl
from jax.experimental.pallas import tpu as pltpu
from jax.experimental.pallas import tpu_sc as plsc

_FM = 128
_NV = 10000
_NCON = 10000
_NE = 320000
_OUT = 16

_NSC = 2
_NSUB = 16
_EPS = _NE // _NSUB
_CH = 64
_NCHUNK = 314
_EPSP = _NCHUNK * _CH
_RPC = 5120
_NPAD = _NSC * _RPC
_ACC = _RPC + 8
_RPS = _RPC // _NSUB

_DNUMS = lax.GatherDimensionNumbers(
    offset_dims=(), collapsed_slice_dims=(0,), start_index_map=(0,))


def _segsum_body(table, gidx, sidx, vals, zeros, out,
                 gidx_v, sidx_v, vals_v, rows0, rows1, sloc, acc,
                 gsem0, gsem1):
    c = lax.axis_index("c")
    s = lax.axis_index("s")
    base = c * _RPC
    rows = (rows0, rows1)
    gsem = (gsem0, gsem1)

    pltpu.sync_copy(zeros, acc.at[pl.ds(s * _RPS, _RPS)])
    pltpu.sync_copy(gidx.at[s], gidx_v)
    pltpu.sync_copy(sidx.at[s], sidx_v)
    pltpu.sync_copy(vals.at[s], vals_v)
    plsc.subcore_barrier()
    pltpu.async_copy(table.at[gidx_v.at[pl.ds(0, _CH)]], rows0, gsem0)

    def step(t, p):
        q = 1 - p
        tn = jnp.minimum(t + 1, _NCHUNK - 1)
        pltpu.async_copy(table.at[gidx_v.at[pl.ds(tn * _CH, _CH)]],
                         rows[q], gsem[q])
        pltpu.make_async_copy(table.at[pl.ds(0, _CH)], rows[p],
                              gsem[p]).wait()

        def grp(g, carry2):
            i16 = sidx_v[pl.ds(t * _CH + g * 16, 16)] - base
            ok = jnp.logical_and(i16 >= 0, i16 < _RPC)
            sloc[pl.ds(g * 16, 16)] = jnp.where(ok, i16, _RPC)
            v16 = vals_v[pl.ds(t * _CH + g * 16, 16)]
            for j in range(16):
                b16 = lax.gather(
                    v16, jnp.full((16, 1), j, jnp.int32), _DNUMS, (1,),
                    mode=lax.GatherScatterMode.PROMISE_IN_BOUNDS)
                r = g * 16 + j
                for k in range(8):
                    rows[p][r, pl.ds(k * 16, 16)] = (
                        rows[p][r, pl.ds(k * 16, 16)] * b16)
            return carry2

        lax.fori_loop(0, _CH // 16, grp, 0)
        pltpu.sync_copy(rows[p], acc.at[sloc], add=True)

    def duo(t2, carry):
        step(t2 * 2, 0)
        step(t2 * 2 + 1, 1)
        return carry

    lax.fori_loop(0, _NCHUNK // 2, duo, 0)
    pltpu.make_async_copy(table.at[pl.ds(0, _CH)], rows0, gsem0).wait()
    plsc.subcore_barrier()
    pltpu.sync_copy(acc.at[pl.ds(s * _RPS, _RPS)],
                    out.at[pl.ds(base + s * _RPS, _RPS)])


_segsum_sc = functools.partial(
    pl.kernel,
    mesh=plsc.VectorSubcoreMesh(core_axis_name="c", subcore_axis_name="s"),
    out_type=jax.ShapeDtypeStruct((_NPAD, _FM), jnp.float32),
    scratch_types=[
        pltpu.VMEM((_EPSP,), jnp.int32),
        pltpu.VMEM((_EPSP,), jnp.int32),
        pltpu.VMEM((_EPSP,), jnp.float32),
        pltpu.VMEM((_CH, _FM), jnp.float32),
        pltpu.VMEM((_CH, _FM), jnp.float32),
        pltpu.VMEM((_CH,), jnp.int32),
        pltpu.VMEM_SHARED((_ACC, _FM), jnp.float32),
        pltpu.SemaphoreType.DMA,
        pltpu.SemaphoreType.DMA,
    ],
)(_segsum_body)


def _mm(x, W, b):
    return jnp.dot(x, W, preferred_element_type=jnp.float32) + b


def _pairnorm(y):
    y = y - jnp.mean(y, axis=0, keepdims=True)
    rn = jnp.sqrt(1e-6 + jnp.mean(jnp.sum(y * y, axis=1)))
    return y / rn


def _q_body(var_ref, qc1W, qc1b, qc2W, qc2b, qo1W, qo1b, qo2W, qo2b, objm,
            cq_ref, oq_ref):
    v = var_ref[...]
    h = jnp.maximum(_mm(v, qc1W[...], qc1b[...]), 0.0)
    cq_ref[...] = _mm(h, qc2W[...], qc2b[...])
    h = jnp.maximum(_mm(v, qo1W[...], qo1b[...]), 0.0)
    oq_ref[...] = _mm(h, qo2W[...], qo2b[...]) * objm[...]


def _c_body(con_ref, v2c_ref, cv_ref, cu1aW, cu1bW, cu1b, cu2W, cu2b,
            out_ref):
    v2c = v2c_ref[:_NCON]
    loss = jnp.maximum(v2c - cv_ref[...], 0.0)
    h = jnp.dot(con_ref[...], cu1aW[...], preferred_element_type=jnp.float32)
    h = h + jnp.dot(loss, cu1bW[...], preferred_element_type=jnp.float32)
    h = jnp.maximum(h + cu1b[...], 0.0)
    out_ref[...] = _pairnorm(_mm(h, cu2W[...], cu2b[...]))


def _v_body(var_ref, c2v_ref, oq_ref, vu1aW, vu1bW, vu1cW, vu1b, vu2W, vu2b,
            o1W, o1b, o2W, o2b, nim_ref, newvar_ref, out_ref):
    h = jnp.dot(var_ref[...], vu1aW[...], preferred_element_type=jnp.float32)
    h = h + jnp.dot(c2v_ref[:_NV], vu1bW[...],
                    preferred_element_type=jnp.float32)
    h = h + jnp.dot(oq_ref[...], vu1cW[...], preferred_element_type=jnp.float32)
    h = jnp.maximum(h + vu1b[...], 0.0)
    y = _pairnorm(_mm(h, vu2W[...], vu2b[...]))
    newvar_ref[...] = y
    h = jnp.maximum(_mm(y, o1W[...], o1b[...]), 0.0)
    out_ref[...] = jax.nn.sigmoid(_mm(h, o2W[...], o2b[...]) + nim_ref[...])


def _f32(shape):
    return jax.ShapeDtypeStruct(shape, jnp.float32)


def _pad_edges(x):
    return jnp.pad(x.reshape(_NSUB, _EPS), ((0, 0), (0, _EPSP - _EPS)))


def kernel(edge_index, edge_vals, objective_multipliers, const_values,
           integer_mask, cu1_W, cu1_b, cu2_W, cu2_b, qc1_W, qc1_b, qc2_W,
           qc2_b, qo1_W, qo1_b, qo2_W, qo2_b, vu1_W, vu1_b, vu2_W, vu2_b,
           o1_W, o1_b, o2_W, o2_b):
    src2 = _pad_edges(edge_index[0])
    dst2 = _pad_edges(edge_index[1])
    vals2 = _pad_edges(edge_vals)
    zeros = jnp.zeros((_RPS, _FM), dtype=jnp.float32)
    objm = objective_multipliers[:, None]
    cv = const_values[:, None]
    im = integer_mask[:, None]
    b = {n: v.reshape(1, -1) for n, v in (
        ("cu1", cu1_b), ("cu2", cu2_b), ("qc1", qc1_b), ("qc2", qc2_b),
        ("qo1", qo1_b), ("qo2", qo2_b), ("vu1", vu1_b), ("vu2", vu2_b),
        ("o1", o1_b), ("o2", o2_b))}

    q_call = pl.pallas_call(
        _q_body, out_shape=[_f32((_NV, _FM)), _f32((_NV, _FM))])
    c_call = pl.pallas_call(_c_body, out_shape=_f32((_NCON, _FM)))
    v_call = pl.pallas_call(
        _v_body, out_shape=[_f32((_NV, _FM)), _f32((_NV, _OUT))])

    variables = jnp.ones((_NV, _FM), dtype=jnp.float32)
    constraints = jnp.ones((_NCON, _FM), dtype=jnp.float32)
    nkey = jax.random.key(42)
    outputs = []
    for i in range(3):
        cq, oq = q_call(variables, qc1_W, b["qc1"], qc2_W, b["qc2"],
                        qo1_W, b["qo1"], qo2_W, b["qo2"], objm)
        v2c = _segsum_sc(cq, src2, dst2, vals2, zeros)
        constraints = c_call(constraints, v2c, cv, cu1_W[:_FM],
                             cu1_W[_FM:], b["cu1"], cu2_W, b["cu2"])
        c2v = _segsum_sc(constraints, dst2, src2, vals2, zeros)
        noise = jax.random.normal(jax.random.fold_in(nkey, i), (_NV, _OUT),
                                  dtype=jnp.float32)
        variables, out_i = v_call(
            variables, c2v, oq, vu1_W[:_FM], vu1_W[_FM:2 * _FM],
            vu1_W[2 * _FM:], b["vu1"], vu2_W, b["vu2"], o1_W, b["o1"],
            o2_W, b["o2"], noise * im)
        outputs.append(out_i)
    return jnp.stack(outputs)
